# Initial kernel scaffold; baseline (speedup 1.0000x reference)
import jax
import jax.numpy as jnp
from jax.experimental import pallas as pl


def kernel(x, y, emb1, emb2, W1, W2):
    raise NotImplementedError("write your pallas kernel here")


# speedup vs baseline: 5.8582x; 5.8582x over previous
import functools

import jax
import jax.numpy as jnp
from jax import lax
from jax.experimental import pallas as pl
from jax.experimental.pallas import tpu as pltpu
from jax.experimental.pallas import tpu_sc as plsc

_N = 819200
_IDX_LANES = 128
_K = 8
_NW = 32
_ROWS_PER_W = _N // (_IDX_LANES * _NW)
_ITERS = _ROWS_PER_W // _K


def _sc_gather(table, idx2d, emb_dim):
    mesh = plsc.VectorSubcoreMesh(core_axis_name="c", subcore_axis_name="s")

    @functools.partial(
        pl.kernel,
        out_type=jax.ShapeDtypeStruct((_N // _IDX_LANES, _IDX_LANES, emb_dim),
                                      jnp.float32),
        mesh=mesh,
        scratch_types=[
            pltpu.VMEM((_K, _IDX_LANES), jnp.int32),
            pltpu.VMEM((_K, _IDX_LANES, emb_dim), jnp.float32),
            pltpu.SemaphoreType.DMA,
        ],
        compiler_params=pltpu.CompilerParams(use_tc_tiling_on_sc=False),
    )
    def gather_kernel(tab_ref, idx_ref, out_ref, idx_v, rows_v, sem):
        wid = lax.axis_index("s") * 2 + lax.axis_index("c")
        row0 = wid * _ROWS_PER_W

        def body(i, carry):
            base = row0 + i * _K
            pltpu.sync_copy(idx_ref.at[pl.ds(base, _K)], idx_v)
            descs = [
                pltpu.async_copy(tab_ref.at[idx_v.at[j]], rows_v.at[j], sem)
                for j in range(_K)
            ]
            for d in descs:
                d.wait()
            pltpu.sync_copy(rows_v, out_ref.at[pl.ds(base, _K)])
            return carry

        lax.fori_loop(0, _ITERS, body, 0)

    return gather_kernel(table, idx2d)


def _tc_linear_relu(rows, idx_col, w_t):
    n, d = rows.shape
    out_dim = w_t.shape[1]
    blk = 1024
    grid = n // blk

    def mm_kernel(g_ref, i_ref, w_ref, o_ref):
        m = i_ref[...] != 0
        g = jnp.where(m, g_ref[...], 0.0)
        o_ref[...] = jnp.maximum(
            jnp.dot(g, w_ref[...], preferred_element_type=jnp.float32), 0.0)

    return pl.pallas_call(
        mm_kernel,
        grid=(grid,),
        in_specs=[
            pl.BlockSpec((blk, d), lambda i: (i, 0)),
            pl.BlockSpec((blk, 1), lambda i: (i, 0)),
            pl.BlockSpec((d, out_dim), lambda i: (0, 0)),
        ],
        out_specs=pl.BlockSpec((blk, out_dim), lambda i: (i, 0)),
        out_shape=jax.ShapeDtypeStruct((n, out_dim), jnp.float32),
    )(rows, idx_col, w_t)


def kernel(x, y, emb1, emb2, W1, W2):
    b, l = x.shape
    out_dim = W1.shape[0]

    x2d = x.reshape(_N // _IDX_LANES, _IDX_LANES)
    y2d = y.reshape(_N // _IDX_LANES, _IDX_LANES)

    g1 = _sc_gather(emb1, x2d, emb1.shape[1]).reshape(_N, emb1.shape[1])
    g2 = _sc_gather(emb2, y2d, emb2.shape[1]).reshape(_N, emb2.shape[1])

    o1 = _tc_linear_relu(g1, x.reshape(_N, 1), W1.T)
    o2 = _tc_linear_relu(g2, y.reshape(_N, 1), W2.T)

    return (o1.reshape(b, l, out_dim), o2.reshape(b, l, out_dim))


# speedup vs baseline: 8.8789x; 1.5156x over previous
import functools

import jax
import jax.numpy as jnp
from jax import lax
from jax.experimental import pallas as pl
from jax.experimental.pallas import tpu as pltpu
from jax.experimental.pallas import tpu_sc as plsc

_B = 16384
_L = 50
_N = _B * _L
_PACK = 4
_LINES = _N // _PACK
_CHUNK_LINES = 128
_NCHUNKS = _LINES // _CHUNK_LINES
_NW = 32
_CHUNKS_PER_W = _NCHUNKS // _NW


def _sc_gather_packed(emb1, emb2, xg, yg):
    mesh = plsc.VectorSubcoreMesh(core_axis_name="c", subcore_axis_name="s")

    @functools.partial(
        pl.kernel,
        out_type=(
            jax.ShapeDtypeStruct((_LINES, 128), jnp.float32),
            jax.ShapeDtypeStruct((_LINES, 128), jnp.float32),
        ),
        mesh=mesh,
        scratch_types=[
            pltpu.VMEM((_PACK, _CHUNK_LINES), jnp.int32),
            pltpu.VMEM((_PACK, _CHUNK_LINES, 32), jnp.float32),
            pltpu.SemaphoreType.DMA,
        ],
        compiler_params=pltpu.CompilerParams(use_tc_tiling_on_sc=False),
    )
    def gather_kernel(t1_ref, t2_ref, xg_ref, yg_ref, o1_ref, o2_ref,
                      idx_v, rows_v, sem):
        wid = lax.axis_index("s") * 2 + lax.axis_index("c")
        chunk0 = wid * _CHUNKS_PER_W

        def make_body(tab_ref, ig_ref, out_ref):
            def body(i, carry):
                line0 = (chunk0 + i) * _CHUNK_LINES
                pltpu.sync_copy(
                    ig_ref.at[:, pl.ds(line0, _CHUNK_LINES)], idx_v)
                descs = [
                    pltpu.async_copy(
                        tab_ref.at[idx_v.at[j]], rows_v.at[j], sem)
                    for j in range(_PACK)
                ]
                for d in descs:
                    d.wait()
                for j in range(_PACK):
                    pltpu.sync_copy(
                        rows_v.at[j],
                        out_ref.at[pl.ds(line0, _CHUNK_LINES),
                                   pl.ds(32 * j, 32)])
                return carry
            return body

        lax.fori_loop(0, _CHUNKS_PER_W, make_body(t1_ref, xg_ref, o1_ref), 0)
        lax.fori_loop(0, _CHUNKS_PER_W, make_body(t2_ref, yg_ref, o2_ref), 0)

    return gather_kernel(emb1, emb2, xg, yg)


def _tc_linear_relu_packed(gw, w_t):
    out_dim = w_t.shape[1]
    lines_per_blk = 1600
    grid = _LINES // lines_per_blk

    bd = jnp.zeros((_PACK, 128, out_dim), jnp.float32)
    for j in range(_PACK):
        bd = bd.at[j, 32 * j:32 * (j + 1), :].set(w_t)

    def mm_kernel(g_ref, w_ref, o_ref):
        gwb = g_ref[...]
        for j in range(_PACK):
            o_ref[:, pl.ds(out_dim * j, out_dim)] = jnp.maximum(
                jnp.dot(gwb, w_ref[j], preferred_element_type=jnp.float32),
                0.0)

    return pl.pallas_call(
        mm_kernel,
        grid=(grid,),
        in_specs=[
            pl.BlockSpec((lines_per_blk, 128), lambda i: (i, 0)),
            pl.BlockSpec((_PACK, 128, out_dim), lambda i: (0, 0, 0)),
        ],
        out_specs=pl.BlockSpec((lines_per_blk, _PACK * out_dim),
                               lambda i: (i, 0)),
        out_shape=jax.ShapeDtypeStruct((_LINES, _PACK * out_dim),
                                       jnp.float32),
    )(gw, bd)


def kernel(x, y, emb1, emb2, W1, W2):
    out_dim = W1.shape[0]
    xg = x.reshape(_LINES, _PACK).T
    yg = y.reshape(_LINES, _PACK).T

    g1, g2 = _sc_gather_packed(emb1, emb2, xg, yg)

    p1 = _tc_linear_relu_packed(g1, W1.T)
    p2 = _tc_linear_relu_packed(g2, W2.T)

    o1 = jnp.where((x != 0)[:, :, None],
                   p1.reshape(_B, _L, out_dim), 0.0)
    o2 = jnp.where((y != 0)[:, :, None],
                   p2.reshape(_B, _L, out_dim), 0.0)
    return (o1, o2)
